# baseline (speedup 1.0000x reference)
import functools

import jax
import jax.numpy as jnp
from jax import lax
from jax.experimental import pallas as pl
from jax.experimental.pallas import tpu as pltpu
from jax.experimental.pallas import tpu_sc as plsc

N = 50000
D = 256
NSEG = 128
NCORES = 2
NSUB = 16
NW = NCORES * NSUB
SEG_PER_W = NSEG // NW
CH = 128
NG = D // 16
U = 16

_NEG_INF = float("-inf")
_POS_INF = float("inf")


def _pool_kernel(x_hbm, batch_hbm, out_hbm, batch_v, xbuf0, xbuf1, accbuf,
                 offs_s, sem0, sem1):
    wid = lax.axis_index("s") * NCORES + lax.axis_index("c")
    s_base = wid * SEG_PER_W

    pltpu.sync_copy(batch_hbm, batch_v.at[pl.ds(0, N)])
    batch_v[pl.ds(N, 16)] = jnp.full((16,), NSEG, jnp.int32)

    def lower_bound(target):
        def body(_, lohi):
            lo, hi = lohi
            mid = (lo + hi) // 2
            v = batch_v[pl.ds(mid, 16)][0]
            pred = v < target
            return jnp.where(pred, mid + 1, lo), jnp.where(pred, hi, mid)

        lo, _ = lax.fori_loop(0, 16, body, (jnp.int32(0), jnp.int32(N)))
        return lo

    for k in range(SEG_PER_W + 1):
        offs_s[k] = lower_bound(s_base + k)

    def acc_rows(xbuf, ab, base, nrows):
        nrows = jnp.maximum(nrows, 0)
        for g in range(NG):
            mx0 = accbuf[pl.ds(ab + g * 16, 16)]
            mn0 = accbuf[pl.ds(ab + D + g * 16, 16)]
            sm0 = accbuf[pl.ds(ab + 2 * D + g * 16, 16)]
            ninf = jnp.full((16,), _NEG_INF, jnp.float32)
            pinf = jnp.full((16,), _POS_INF, jnp.float32)
            zero = jnp.zeros((16,), jnp.float32)
            init = (base, mx0, ninf, ninf, ninf, mn0, pinf, pinf, pinf,
                    sm0, zero, zero, zero)

            def bodyU(j, c):
                r = c[0]
                v = [xbuf[r + t, pl.ds(g * 16, 16)] for t in range(U)]
                mx = [c[1 + t] for t in range(4)]
                mn = [c[5 + t] for t in range(4)]
                sm = [c[9 + t] for t in range(4)]
                for t in range(U):
                    mx[t % 4] = jnp.maximum(mx[t % 4], v[t])
                    mn[t % 4] = jnp.minimum(mn[t % 4], v[t])
                    sm[t % 4] = sm[t % 4] + v[t]
                return tuple([r + U] + mx + mn + sm)

            nU = nrows // U
            c = lax.fori_loop(0, nU, bodyU, init)

            def body1(_, c4):
                r, mx, mn, sm = c4
                v = xbuf[r, pl.ds(g * 16, 16)]
                return (r + 1, jnp.maximum(mx, v), jnp.minimum(mn, v),
                        sm + v)

            mx = jnp.maximum(jnp.maximum(c[1], c[2]), jnp.maximum(c[3], c[4]))
            mn = jnp.minimum(jnp.minimum(c[5], c[6]), jnp.minimum(c[7], c[8]))
            sm = (c[9] + c[10]) + (c[11] + c[12])
            _, mx, mn, sm = lax.fori_loop(0, nrows - nU * U, body1,
                                          (c[0], mx, mn, sm))
            accbuf[pl.ds(ab + g * 16, 16)] = mx
            accbuf[pl.ds(ab + D + g * 16, 16)] = mn
            accbuf[pl.ds(ab + 2 * D + g * 16, 16)] = sm

    def init_body(k, _):
        ab = k * 3 * D
        for g in range(NG):
            accbuf[pl.ds(ab + g * 16, 16)] = jnp.full((16,), _NEG_INF, jnp.float32)
            accbuf[pl.ds(ab + D + g * 16, 16)] = jnp.full((16,), _POS_INF, jnp.float32)
            accbuf[pl.ds(ab + 2 * D + g * 16, 16)] = jnp.zeros((16,), jnp.float32)
        return 0

    lax.fori_loop(0, SEG_PER_W, init_body, 0)

    o_beg = offs_s[0]
    o_end = offs_s[SEG_PER_W]
    a_s = (o_beg // 8) * 8
    nch = (o_end - a_s + CH - 1) // CH

    def chunk_start(i):
        return pl.multiple_of(jnp.minimum(a_s + i * CH, N - CH), 8)

    def dma_start(i, xbuf, sem):
        pltpu.async_copy(x_hbm.at[pl.ds(chunk_start(i), CH)], xbuf, sem)

    def dma_wait(i, xbuf, sem):
        pltpu.make_async_copy(x_hbm.at[pl.ds(chunk_start(i), CH)], xbuf,
                              sem).wait()

    def process(i, xbuf):
        st = a_s + i * CH
        std = chunk_start(i)

        def seg_k(k, _):
            lo = jnp.maximum(offs_s[k], st)
            hi = jnp.minimum(offs_s[k + 1], st + CH)

            @pl.when(hi > lo)
            def _():
                acc_rows(xbuf, k * 3 * D, lo - std, hi - lo)

            return 0

        lax.fori_loop(0, SEG_PER_W, seg_k, 0)

    @pl.when(nch > 0)
    def _():
        dma_start(0, xbuf0, sem0)

    def pair_body(j, _):
        i0 = 2 * j
        i1 = i0 + 1

        @pl.when(i1 < nch)
        def _():
            dma_start(i1, xbuf1, sem1)

        dma_wait(i0, xbuf0, sem0)
        process(i0, xbuf0)

        @pl.when(i0 + 2 < nch)
        def _():
            dma_start(i0 + 2, xbuf0, sem0)

        @pl.when(i1 < nch)
        def _():
            dma_wait(i1, xbuf1, sem1)
            process(i1, xbuf1)

        return 0

    lax.fori_loop(0, (nch + 1) // 2, pair_body, 0)

    def fin_body(k, _):
        ab = k * 3 * D
        cnt = offs_s[k + 1] - offs_s[k]
        denom = jnp.maximum(cnt.astype(jnp.float32), 1.0)
        for g in range(NG):
            sm = accbuf[pl.ds(ab + 2 * D + g * 16, 16)]
            accbuf[pl.ds(ab + 2 * D + g * 16, 16)] = sm / denom
        pltpu.sync_copy(accbuf.at[pl.ds(ab, 3 * D)], out_hbm.at[s_base + k])
        return 0

    lax.fori_loop(0, SEG_PER_W, fin_body, 0)


def _pool(x2d, batch):
    mesh = plsc.VectorSubcoreMesh(core_axis_name="c", subcore_axis_name="s")
    call = functools.partial(
        pl.kernel,
        mesh=mesh,
        out_type=jax.ShapeDtypeStruct((NSEG, 3 * D), jnp.float32),
        scratch_types=[
            pltpu.VMEM((N + 16,), jnp.int32),
            pltpu.VMEM((CH, D), jnp.float32),
            pltpu.VMEM((CH, D), jnp.float32),
            pltpu.VMEM((SEG_PER_W * 3 * D,), jnp.float32),
            pltpu.SMEM((SEG_PER_W + 1,), jnp.int32),
            pltpu.SemaphoreType.DMA,
            pltpu.SemaphoreType.DMA,
        ],
    )(_pool_kernel)
    return call(x2d, batch)


def _mm_kernel(feat_ref, w_ref, b_ref, out_ref):
    out_ref[...] = (
        jnp.dot(feat_ref[...], w_ref[...], preferred_element_type=jnp.float32)
        + b_ref[...]
    )


def _mm(feat, W, b):
    return pl.pallas_call(
        _mm_kernel,
        out_shape=jax.ShapeDtypeStruct((NSEG, D), jnp.float32),
    )(feat, W, b.reshape(1, D))


def kernel(x, batch, W, b):
    feat = _pool(x, batch.astype(jnp.int32))
    return _mm(feat, W, b)


# speedup vs baseline: 1.2529x; 1.2529x over previous
import functools

import jax
import jax.numpy as jnp
from jax import lax
from jax.experimental import pallas as pl
from jax.experimental.pallas import tpu as pltpu
from jax.experimental.pallas import tpu_sc as plsc

N = 50000
D = 256
NSEG = 128
NCORES = 2
NSUB = 16
NW = NCORES * NSUB
SEG_PER_W = NSEG // NW
CH = 128
NG = D // 16
U = 8

_NEG_INF = float("-inf")
_POS_INF = float("inf")


def _pool_kernel(x_hbm, batch_hbm, out_hbm, batch_v, xbuf0, xbuf1, accbuf,
                 offs_s, sem0, sem1):
    wid = lax.axis_index("s") * NCORES + lax.axis_index("c")
    s_base = wid * SEG_PER_W

    pltpu.sync_copy(batch_hbm, batch_v.at[pl.ds(0, N)])
    batch_v[pl.ds(N, 16)] = jnp.full((16,), NSEG, jnp.int32)

    def lower_bound(target):
        def body(_, lohi):
            lo, hi = lohi
            mid = (lo + hi) // 2
            v = batch_v[pl.ds(mid, 16)][0]
            pred = v < target
            return jnp.where(pred, mid + 1, lo), jnp.where(pred, hi, mid)

        lo, _ = lax.fori_loop(0, 16, body, (jnp.int32(0), jnp.int32(N)))
        return lo

    for k in range(SEG_PER_W + 1):
        offs_s[k] = lower_bound(s_base + k)

    def acc_rows(xbuf, ab, base, nrows):
        nrows = jnp.maximum(nrows, 0)
        for g in range(NG):
            mx0 = accbuf[pl.ds(ab + g * 16, 16)]
            mn0 = accbuf[pl.ds(ab + D + g * 16, 16)]
            sm0 = accbuf[pl.ds(ab + 2 * D + g * 16, 16)]
            ninf = jnp.full((16,), _NEG_INF, jnp.float32)
            pinf = jnp.full((16,), _POS_INF, jnp.float32)
            zero = jnp.zeros((16,), jnp.float32)
            init = (base, mx0, ninf, ninf, ninf, mn0, pinf, pinf, pinf,
                    sm0, zero, zero, zero)

            def bodyU(j, c):
                r = c[0]
                v = [xbuf[r + t, pl.ds(g * 16, 16)] for t in range(U)]
                mx = [c[1 + t] for t in range(4)]
                mn = [c[5 + t] for t in range(4)]
                sm = [c[9 + t] for t in range(4)]
                for t in range(U):
                    mx[t % 4] = jnp.maximum(mx[t % 4], v[t])
                    mn[t % 4] = jnp.minimum(mn[t % 4], v[t])
                    sm[t % 4] = sm[t % 4] + v[t]
                return tuple([r + U] + mx + mn + sm)

            nU = nrows // U
            c = lax.fori_loop(0, nU, bodyU, init)

            def body1(_, c4):
                r, mx, mn, sm = c4
                v = xbuf[r, pl.ds(g * 16, 16)]
                return (r + 1, jnp.maximum(mx, v), jnp.minimum(mn, v),
                        sm + v)

            mx = jnp.maximum(jnp.maximum(c[1], c[2]), jnp.maximum(c[3], c[4]))
            mn = jnp.minimum(jnp.minimum(c[5], c[6]), jnp.minimum(c[7], c[8]))
            sm = (c[9] + c[10]) + (c[11] + c[12])
            _, mx, mn, sm = lax.fori_loop(0, nrows - nU * U, body1,
                                          (c[0], mx, mn, sm))
            accbuf[pl.ds(ab + g * 16, 16)] = mx
            accbuf[pl.ds(ab + D + g * 16, 16)] = mn
            accbuf[pl.ds(ab + 2 * D + g * 16, 16)] = sm

    def init_body(k, _):
        ab = k * 3 * D
        for g in range(NG):
            accbuf[pl.ds(ab + g * 16, 16)] = jnp.full((16,), _NEG_INF, jnp.float32)
            accbuf[pl.ds(ab + D + g * 16, 16)] = jnp.full((16,), _POS_INF, jnp.float32)
            accbuf[pl.ds(ab + 2 * D + g * 16, 16)] = jnp.zeros((16,), jnp.float32)
        return 0

    lax.fori_loop(0, SEG_PER_W, init_body, 0)

    o_beg = offs_s[0]
    o_end = offs_s[SEG_PER_W]
    a_s = (o_beg // 8) * 8
    nch = (o_end - a_s + CH - 1) // CH

    def chunk_start(i):
        return pl.multiple_of(jnp.minimum(a_s + i * CH, N - CH), 8)

    def dma_start(i, xbuf, sem):
        pltpu.async_copy(x_hbm.at[pl.ds(chunk_start(i), CH)], xbuf, sem)

    def dma_wait(i, xbuf, sem):
        pltpu.make_async_copy(x_hbm.at[pl.ds(chunk_start(i), CH)], xbuf,
                              sem).wait()

    def process(i, xbuf):
        st = a_s + i * CH
        std = chunk_start(i)

        def seg_k(k, _):
            lo = jnp.maximum(offs_s[k], st)
            hi = jnp.minimum(offs_s[k + 1], st + CH)

            @pl.when(hi > lo)
            def _():
                acc_rows(xbuf, k * 3 * D, lo - std, hi - lo)

            return 0

        lax.fori_loop(0, SEG_PER_W, seg_k, 0)

    @pl.when(nch > 0)
    def _():
        dma_start(0, xbuf0, sem0)

    def pair_body(j, _):
        i0 = 2 * j
        i1 = i0 + 1

        @pl.when(i1 < nch)
        def _():
            dma_start(i1, xbuf1, sem1)

        dma_wait(i0, xbuf0, sem0)
        process(i0, xbuf0)

        @pl.when(i0 + 2 < nch)
        def _():
            dma_start(i0 + 2, xbuf0, sem0)

        @pl.when(i1 < nch)
        def _():
            dma_wait(i1, xbuf1, sem1)
            process(i1, xbuf1)

        return 0

    lax.fori_loop(0, (nch + 1) // 2, pair_body, 0)

    def fin_body(k, _):
        ab = k * 3 * D
        cnt = offs_s[k + 1] - offs_s[k]
        denom = jnp.maximum(cnt.astype(jnp.float32), 1.0)
        for g in range(NG):
            sm = accbuf[pl.ds(ab + 2 * D + g * 16, 16)]
            accbuf[pl.ds(ab + 2 * D + g * 16, 16)] = sm / denom
        pltpu.sync_copy(accbuf.at[pl.ds(ab, 3 * D)], out_hbm.at[s_base + k])
        return 0

    lax.fori_loop(0, SEG_PER_W, fin_body, 0)


def _pool(x2d, batch):
    mesh = plsc.VectorSubcoreMesh(core_axis_name="c", subcore_axis_name="s")
    call = functools.partial(
        pl.kernel,
        mesh=mesh,
        out_type=jax.ShapeDtypeStruct((NSEG, 3 * D), jnp.float32),
        scratch_types=[
            pltpu.VMEM((N + 16,), jnp.int32),
            pltpu.VMEM((CH, D), jnp.float32),
            pltpu.VMEM((CH, D), jnp.float32),
            pltpu.VMEM((SEG_PER_W * 3 * D,), jnp.float32),
            pltpu.SMEM((SEG_PER_W + 1,), jnp.int32),
            pltpu.SemaphoreType.DMA,
            pltpu.SemaphoreType.DMA,
        ],
    )(_pool_kernel)
    return call(x2d, batch)


def _mm_kernel(feat_ref, w_ref, b_ref, out_ref):
    out_ref[...] = (
        jnp.dot(feat_ref[...], w_ref[...], preferred_element_type=jnp.float32)
        + b_ref[...]
    )


def _mm(feat, W, b):
    return pl.pallas_call(
        _mm_kernel,
        out_shape=jax.ShapeDtypeStruct((NSEG, D), jnp.float32),
    )(feat, W, b.reshape(1, D))


def kernel(x, batch, W, b):
    feat = _pool(x, batch.astype(jnp.int32))
    return _mm(feat, W, b)


# speedup vs baseline: 1.9341x; 1.5437x over previous
import functools

import jax
import jax.numpy as jnp
from jax import lax
from jax.experimental import pallas as pl
from jax.experimental.pallas import tpu as pltpu
from jax.experimental.pallas import tpu_sc as plsc

N = 50000
D = 256
NSEG = 128
NCORES = 2
NSUB = 16
NW = NCORES * NSUB
SEG_PER_W = NSEG // NW
CH = 128
NG = D // 16
GH = 8

_NEG_INF = float("-inf")
_POS_INF = float("inf")


def _pool_kernel(x_hbm, batch_hbm, out_hbm, batch_v, xbuf0, xbuf1, accbuf,
                 offs_s, sem0, sem1):
    wid = lax.axis_index("s") * NCORES + lax.axis_index("c")
    s_base = wid * SEG_PER_W

    pltpu.sync_copy(batch_hbm, batch_v.at[pl.ds(0, N)])
    batch_v[pl.ds(N, 16)] = jnp.full((16,), NSEG, jnp.int32)

    def lower_bound(target):
        def body(_, lohi):
            lo, hi = lohi
            mid = (lo + hi) // 2
            v = batch_v[pl.ds(mid, 16)][0]
            pred = v < target
            return jnp.where(pred, mid + 1, lo), jnp.where(pred, hi, mid)

        lo, _ = lax.fori_loop(0, 16, body, (jnp.int32(0), jnp.int32(N)))
        return lo

    for k in range(SEG_PER_W + 1):
        offs_s[k] = lower_bound(s_base + k)

    def acc_rows(xbuf, ab, base, nrows):
        nrows = jnp.maximum(nrows, 0)
        for gh in range(NG // GH):
            g0 = gh * GH
            mx0 = [accbuf[pl.ds(ab + (g0 + t) * 16, 16)] for t in range(GH)]
            mn0 = [accbuf[pl.ds(ab + D + (g0 + t) * 16, 16)]
                   for t in range(GH)]
            sm0 = [accbuf[pl.ds(ab + 2 * D + (g0 + t) * 16, 16)]
                   for t in range(GH)]
            init = tuple([base] + mx0 + mn0 + sm0)

            def body(_, c):
                r = c[0]
                v = [xbuf[r, pl.ds((g0 + t) * 16, 16)] for t in range(GH)]
                mx = [jnp.maximum(c[1 + t], v[t]) for t in range(GH)]
                mn = [jnp.minimum(c[1 + GH + t], v[t]) for t in range(GH)]
                sm = [c[1 + 2 * GH + t] + v[t] for t in range(GH)]
                return tuple([r + 1] + mx + mn + sm)

            c = lax.fori_loop(0, nrows, body, init)
            for t in range(GH):
                accbuf[pl.ds(ab + (g0 + t) * 16, 16)] = c[1 + t]
                accbuf[pl.ds(ab + D + (g0 + t) * 16, 16)] = c[1 + GH + t]
                accbuf[pl.ds(ab + 2 * D + (g0 + t) * 16, 16)] = c[1 + 2 * GH + t]

    def init_body(k, _):
        ab = k * 3 * D
        for g in range(NG):
            accbuf[pl.ds(ab + g * 16, 16)] = jnp.full((16,), _NEG_INF, jnp.float32)
            accbuf[pl.ds(ab + D + g * 16, 16)] = jnp.full((16,), _POS_INF, jnp.float32)
            accbuf[pl.ds(ab + 2 * D + g * 16, 16)] = jnp.zeros((16,), jnp.float32)
        return 0

    lax.fori_loop(0, SEG_PER_W, init_body, 0)

    o_beg = offs_s[0]
    o_end = offs_s[SEG_PER_W]
    a_s = (o_beg // 8) * 8
    nch = (o_end - a_s + CH - 1) // CH

    def chunk_start(i):
        return pl.multiple_of(jnp.minimum(a_s + i * CH, N - CH), 8)

    def dma_start(i, xbuf, sem):
        pltpu.async_copy(x_hbm.at[pl.ds(chunk_start(i), CH)], xbuf, sem)

    def dma_wait(i, xbuf, sem):
        pltpu.make_async_copy(x_hbm.at[pl.ds(chunk_start(i), CH)], xbuf,
                              sem).wait()

    def process(i, xbuf):
        st = a_s + i * CH
        std = chunk_start(i)

        def seg_k(k, _):
            lo = jnp.maximum(offs_s[k], st)
            hi = jnp.minimum(offs_s[k + 1], st + CH)

            @pl.when(hi > lo)
            def _():
                acc_rows(xbuf, k * 3 * D, lo - std, hi - lo)

            return 0

        lax.fori_loop(0, SEG_PER_W, seg_k, 0)

    @pl.when(nch > 0)
    def _():
        dma_start(0, xbuf0, sem0)

    def pair_body(j, _):
        i0 = 2 * j
        i1 = i0 + 1

        @pl.when(i1 < nch)
        def _():
            dma_start(i1, xbuf1, sem1)

        dma_wait(i0, xbuf0, sem0)
        process(i0, xbuf0)

        @pl.when(i0 + 2 < nch)
        def _():
            dma_start(i0 + 2, xbuf0, sem0)

        @pl.when(i1 < nch)
        def _():
            dma_wait(i1, xbuf1, sem1)
            process(i1, xbuf1)

        return 0

    lax.fori_loop(0, (nch + 1) // 2, pair_body, 0)

    def fin_body(k, _):
        ab = k * 3 * D
        cnt = offs_s[k + 1] - offs_s[k]
        denom = jnp.maximum(cnt.astype(jnp.float32), 1.0)
        for g in range(NG):
            sm = accbuf[pl.ds(ab + 2 * D + g * 16, 16)]
            accbuf[pl.ds(ab + 2 * D + g * 16, 16)] = sm / denom
        pltpu.sync_copy(accbuf.at[pl.ds(ab, 3 * D)], out_hbm.at[s_base + k])
        return 0

    lax.fori_loop(0, SEG_PER_W, fin_body, 0)


def _pool(x2d, batch):
    mesh = plsc.VectorSubcoreMesh(core_axis_name="c", subcore_axis_name="s")
    call = functools.partial(
        pl.kernel,
        mesh=mesh,
        out_type=jax.ShapeDtypeStruct((NSEG, 3 * D), jnp.float32),
        scratch_types=[
            pltpu.VMEM((N + 16,), jnp.int32),
            pltpu.VMEM((CH, D), jnp.float32),
            pltpu.VMEM((CH, D), jnp.float32),
            pltpu.VMEM((SEG_PER_W * 3 * D,), jnp.float32),
            pltpu.SMEM((SEG_PER_W + 1,), jnp.int32),
            pltpu.SemaphoreType.DMA,
            pltpu.SemaphoreType.DMA,
        ],
    )(_pool_kernel)
    return call(x2d, batch)


def _mm_kernel(feat_ref, w_ref, b_ref, out_ref):
    out_ref[...] = (
        jnp.dot(feat_ref[...], w_ref[...], preferred_element_type=jnp.float32)
        + b_ref[...]
    )


def _mm(feat, W, b):
    return pl.pallas_call(
        _mm_kernel,
        out_shape=jax.ShapeDtypeStruct((NSEG, D), jnp.float32),
    )(feat, W, b.reshape(1, D))


def kernel(x, batch, W, b):
    feat = _pool(x, batch.astype(jnp.int32))
    return _mm(feat, W, b)


# speedup vs baseline: 1.9431x; 1.0047x over previous
import functools

import jax
import jax.numpy as jnp
from jax import lax
from jax.experimental import pallas as pl
from jax.experimental.pallas import tpu as pltpu
from jax.experimental.pallas import tpu_sc as plsc

N = 50000
D = 256
NSEG = 128
NCORES = 2
NSUB = 16
NW = NCORES * NSUB
SEG_PER_W = NSEG // NW
CH = 144
NG = D // 16
GH = 8

_NEG_INF = float("-inf")
_POS_INF = float("inf")


def _pool_kernel(x_hbm, batch_hbm, out_hbm, batch_v, xbuf0, xbuf1, accbuf,
                 offs_s, sem0, sem1):
    wid = lax.axis_index("s") * NCORES + lax.axis_index("c")
    s_base = wid * SEG_PER_W

    pltpu.sync_copy(batch_hbm, batch_v.at[pl.ds(0, N)])
    batch_v[pl.ds(N, 16)] = jnp.full((16,), NSEG, jnp.int32)

    def lower_bound(target):
        def body(_, lohi):
            lo, hi = lohi
            mid = (lo + hi) // 2
            v = batch_v[pl.ds(mid, 16)][0]
            pred = v < target
            return jnp.where(pred, mid + 1, lo), jnp.where(pred, hi, mid)

        lo, _ = lax.fori_loop(0, 16, body, (jnp.int32(0), jnp.int32(N)))
        return lo

    for k in range(SEG_PER_W + 1):
        offs_s[k] = lower_bound(s_base + k)

    def acc_rows(xbuf, ab, base, nrows):
        nrows = jnp.maximum(nrows, 0)
        for gh in range(NG // GH):
            g0 = gh * GH
            mx0 = [accbuf[pl.ds(ab + (g0 + t) * 16, 16)] for t in range(GH)]
            mn0 = [accbuf[pl.ds(ab + D + (g0 + t) * 16, 16)]
                   for t in range(GH)]
            sm0 = [accbuf[pl.ds(ab + 2 * D + (g0 + t) * 16, 16)]
                   for t in range(GH)]
            init = tuple([base] + mx0 + mn0 + sm0)

            def body(_, c):
                r = c[0]
                v = [xbuf[r, pl.ds((g0 + t) * 16, 16)] for t in range(GH)]
                mx = [jnp.maximum(c[1 + t], v[t]) for t in range(GH)]
                mn = [jnp.minimum(c[1 + GH + t], v[t]) for t in range(GH)]
                sm = [c[1 + 2 * GH + t] + v[t] for t in range(GH)]
                return tuple([r + 1] + mx + mn + sm)

            c = lax.fori_loop(0, nrows, body, init)
            for t in range(GH):
                accbuf[pl.ds(ab + (g0 + t) * 16, 16)] = c[1 + t]
                accbuf[pl.ds(ab + D + (g0 + t) * 16, 16)] = c[1 + GH + t]
                accbuf[pl.ds(ab + 2 * D + (g0 + t) * 16, 16)] = c[1 + 2 * GH + t]

    def init_body(k, _):
        ab = k * 3 * D
        for g in range(NG):
            accbuf[pl.ds(ab + g * 16, 16)] = jnp.full((16,), _NEG_INF, jnp.float32)
            accbuf[pl.ds(ab + D + g * 16, 16)] = jnp.full((16,), _POS_INF, jnp.float32)
            accbuf[pl.ds(ab + 2 * D + g * 16, 16)] = jnp.zeros((16,), jnp.float32)
        return 0

    lax.fori_loop(0, SEG_PER_W, init_body, 0)

    o_beg = offs_s[0]
    o_end = offs_s[SEG_PER_W]
    a_s = (o_beg // 8) * 8
    nch = (o_end - a_s + CH - 1) // CH

    def chunk_start(i):
        return pl.multiple_of(jnp.minimum(a_s + i * CH, N - CH), 8)

    def dma_start(i, xbuf, sem):
        pltpu.async_copy(x_hbm.at[pl.ds(chunk_start(i), CH)], xbuf, sem)

    def dma_wait(i, xbuf, sem):
        pltpu.make_async_copy(x_hbm.at[pl.ds(chunk_start(i), CH)], xbuf,
                              sem).wait()

    def process(i, xbuf):
        st = a_s + i * CH
        std = chunk_start(i)

        def seg_k(k, _):
            lo = jnp.maximum(offs_s[k], st)
            hi = jnp.minimum(offs_s[k + 1], st + CH)

            @pl.when(hi > lo)
            def _():
                acc_rows(xbuf, k * 3 * D, lo - std, hi - lo)

            return 0

        lax.fori_loop(0, SEG_PER_W, seg_k, 0)

    @pl.when(nch > 0)
    def _():
        dma_start(0, xbuf0, sem0)

    def pair_body(j, _):
        i0 = 2 * j
        i1 = i0 + 1

        @pl.when(i1 < nch)
        def _():
            dma_start(i1, xbuf1, sem1)

        dma_wait(i0, xbuf0, sem0)
        process(i0, xbuf0)

        @pl.when(i0 + 2 < nch)
        def _():
            dma_start(i0 + 2, xbuf0, sem0)

        @pl.when(i1 < nch)
        def _():
            dma_wait(i1, xbuf1, sem1)
            process(i1, xbuf1)

        return 0

    lax.fori_loop(0, (nch + 1) // 2, pair_body, 0)

    def fin_body(k, _):
        ab = k * 3 * D
        cnt = offs_s[k + 1] - offs_s[k]
        denom = jnp.maximum(cnt.astype(jnp.float32), 1.0)
        for g in range(NG):
            sm = accbuf[pl.ds(ab + 2 * D + g * 16, 16)]
            accbuf[pl.ds(ab + 2 * D + g * 16, 16)] = sm / denom
        pltpu.sync_copy(accbuf.at[pl.ds(ab, 3 * D)], out_hbm.at[s_base + k])
        return 0

    lax.fori_loop(0, SEG_PER_W, fin_body, 0)


def _pool(x2d, batch):
    mesh = plsc.VectorSubcoreMesh(core_axis_name="c", subcore_axis_name="s")
    call = functools.partial(
        pl.kernel,
        mesh=mesh,
        out_type=jax.ShapeDtypeStruct((NSEG, 3 * D), jnp.float32),
        scratch_types=[
            pltpu.VMEM((N + 16,), jnp.int32),
            pltpu.VMEM((CH, D), jnp.float32),
            pltpu.VMEM((CH, D), jnp.float32),
            pltpu.VMEM((SEG_PER_W * 3 * D,), jnp.float32),
            pltpu.SMEM((SEG_PER_W + 1,), jnp.int32),
            pltpu.SemaphoreType.DMA,
            pltpu.SemaphoreType.DMA,
        ],
    )(_pool_kernel)
    return call(x2d, batch)


def _mm_kernel(feat_ref, w_ref, b_ref, out_ref):
    out_ref[...] = (
        jnp.dot(feat_ref[...], w_ref[...], preferred_element_type=jnp.float32)
        + b_ref[...]
    )


def _mm(feat, W, b):
    return pl.pallas_call(
        _mm_kernel,
        out_shape=jax.ShapeDtypeStruct((NSEG, D), jnp.float32),
    )(feat, W, b.reshape(1, D))


def kernel(x, batch, W, b):
    feat = _pool(x, batch.astype(jnp.int32))
    return _mm(feat, W, b)
